# baseline (speedup 1.0000x reference)
import functools
import math

import jax
import jax.numpy as jnp
from jax import lax
from jax.experimental import pallas as pl
from jax.experimental.pallas import tpu as pltpu
from jax.experimental.pallas import tpu_sc as plsc

D_MODEL = 64
SCALE = math.sqrt(D_MODEL)
NUM_CORES = 2
NUM_SUBCORES = 16
NUM_WORKERS = NUM_CORES * NUM_SUBCORES
LANES = 16


def _embed_kernel(n_total: int, chunk: int):
  b_per_w = n_total // NUM_WORKERS
  n_chunks = b_per_w // chunk
  mesh = plsc.VectorSubcoreMesh(core_axis_name="c", subcore_axis_name="s")

  @functools.partial(
      pl.kernel,
      mesh=mesh,
      out_type=jax.ShapeDtypeStruct((n_total, D_MODEL), jnp.float32),
      scratch_types=[
          pltpu.VMEM((chunk,), jnp.int32),
          pltpu.VMEM((chunk, D_MODEL), jnp.float32),
          pltpu.SemaphoreType.DMA,
      ],
      compiler_params=pltpu.CompilerParams(use_tc_tiling_on_sc=False),
  )
  def k(idx_hbm, table_hbm, out_hbm, idx_v, rows_v, sem):
    wid = lax.axis_index("s") * NUM_CORES + lax.axis_index("c")
    base = wid * b_per_w

    def chunk_body(c, carry):
      off = base + c * chunk
      pltpu.sync_copy(idx_hbm.at[pl.ds(off, chunk)], idx_v)
      pltpu.async_copy(table_hbm.at[idx_v], rows_v, sem).wait()

      def scale_row(r, carry2):
        for j in range(D_MODEL // LANES):
          sl = pl.ds(j * LANES, LANES)
          rows_v[r, sl] = rows_v[r, sl] * SCALE
        return carry2

      lax.fori_loop(0, chunk, scale_row, 0, unroll=4)
      pltpu.sync_copy(rows_v, out_hbm.at[pl.ds(off, chunk)])
      return carry

    lax.fori_loop(0, n_chunks, chunk_body, 0)

  return k


def kernel(input, table):
  b0, b1 = input.shape
  n_total = b0 * b1
  idx_flat = input.reshape(n_total).astype(jnp.int32)
  out = _embed_kernel(n_total, chunk=800)(idx_flat, table)
  return out.reshape(b0, b1, D_MODEL)


# speedup vs baseline: 1.0693x; 1.0693x over previous
import functools
import math

import jax
import jax.numpy as jnp
from jax import lax
from jax.experimental import pallas as pl
from jax.experimental.pallas import tpu as pltpu
from jax.experimental.pallas import tpu_sc as plsc

D_MODEL = 64
SCALE = math.sqrt(D_MODEL)
NUM_CORES = 2
NUM_SUBCORES = 16
NUM_WORKERS = NUM_CORES * NUM_SUBCORES
LANES = 16
NBUF = 4
CHUNK = 400


def _embed_kernel(n_total: int):
  b_per_w = n_total // NUM_WORKERS
  n_chunks = b_per_w // CHUNK
  assert n_total % NUM_WORKERS == 0
  assert b_per_w % CHUNK == 0
  assert n_chunks % NBUF == 0 and n_chunks >= 2 * NBUF
  n_groups = n_chunks // NBUF
  mesh = plsc.VectorSubcoreMesh(core_axis_name="c", subcore_axis_name="s")

  @functools.partial(
      pl.kernel,
      mesh=mesh,
      out_type=jax.ShapeDtypeStruct((n_total, D_MODEL), jnp.float32),
      scratch_types=[
          pltpu.VMEM((b_per_w,), jnp.int32),
          *[pltpu.VMEM((CHUNK, D_MODEL), jnp.float32) for _ in range(NBUF)],
          *[pltpu.SemaphoreType.DMA for _ in range(2 * NBUF)],
      ],
      compiler_params=pltpu.CompilerParams(use_tc_tiling_on_sc=False),
  )
  def k(idx_hbm, table_hbm, out_hbm, idx_v, r0, r1, r2, r3,
        g0, g1, g2, g3, s0, s1, s2, s3):
    rows = [r0, r1, r2, r3]
    gsem = [g0, g1, g2, g3]
    ssem = [s0, s1, s2, s3]
    wid = lax.axis_index("s") * NUM_CORES + lax.axis_index("c")
    base = wid * b_per_w
    pltpu.sync_copy(idx_hbm.at[pl.ds(base, b_per_w)], idx_v)

    def start_gather(c, b):
      off = pl.multiple_of(c * CHUNK, 8)
      pltpu.async_copy(table_hbm.at[idx_v.at[pl.ds(off, CHUNK)]],
                       rows[b], gsem[b])

    def wait_gather(b):
      pltpu.make_async_copy(table_hbm.at[idx_v.at[pl.ds(0, CHUNK)]],
                            rows[b], gsem[b]).wait()

    def wait_scatter(b):
      pltpu.make_async_copy(rows[b], out_hbm.at[pl.ds(0, CHUNK)],
                            ssem[b]).wait()

    def scale_scatter(c, b):
      def srow(r, carry):
        for j in range(D_MODEL // LANES):
          sl = pl.ds(j * LANES, LANES)
          rows[b][r, sl] = rows[b][r, sl] * SCALE
        return carry

      lax.fori_loop(0, CHUNK, srow, 0, unroll=8)
      off = pl.multiple_of(base + c * CHUNK, 8)
      pltpu.async_copy(rows[b], out_hbm.at[pl.ds(off, CHUNK)], ssem[b])

    start_gather(0, 0)
    start_gather(1, 1)
    start_gather(2, 2)
    wait_gather(0)
    scale_scatter(0, 0)
    start_gather(3, 3)
    wait_gather(1)
    scale_scatter(1, 1)
    wait_gather(2)
    scale_scatter(2, 2)
    wait_scatter(0)
    start_gather(4, 0)
    wait_gather(3)
    scale_scatter(3, 3)
    wait_scatter(1)
    start_gather(5, 1)

    def group(g, carry):
      c0 = g * NBUF
      for b in range(NBUF):
        c = c0 + b
        pb = (b + 2) % NBUF
        wait_scatter(pb)
        start_gather(c + 2, pb)
        wait_gather(b)
        scale_scatter(c, b)
      return carry

    lax.fori_loop(1, n_groups - 1, group, 0)

    cL = (n_groups - 1) * NBUF
    for b in range(NBUF):
      if b < 2:
        pb = (b + 2) % NBUF
        wait_scatter(pb)
        start_gather(cL + b + 2, pb)
      wait_gather(b)
      scale_scatter(cL + b, b)
    for b in range(NBUF):
      wait_scatter(b)

  return k


def kernel(input, table):
  b0, b1 = input.shape
  n_total = b0 * b1
  idx_flat = input.reshape(n_total).astype(jnp.int32)
  out = _embed_kernel(n_total)(idx_flat, table)
  return out.reshape(b0, b1, D_MODEL)
